# baseline (speedup 1.0000x reference)
import functools

import jax
import jax.numpy as jnp
from jax import lax
from jax.experimental import pallas as pl
from jax.experimental.pallas import tpu as pltpu
from jax.experimental.pallas import tpu_sc as plsc

B = 16384
D = 64
H = 128
NC, NS = 2, 16
NW = NC * NS
BPW = B // NW
ICH = 128
NCH = BPW // ICH

@functools.cache
def _build_sc_gather():
    mesh = plsc.VectorSubcoreMesh(
        core_axis_name="c", subcore_axis_name="s", num_cores=NC, num_subcores=NS
    )

    @functools.partial(
        pl.kernel,
        out_type=(
            jax.ShapeDtypeStruct((B, D), jnp.float32),
            jax.ShapeDtypeStruct((B, D), jnp.float32),
        ),
        mesh=mesh,
        compiler_params=pltpu.CompilerParams(use_tc_tiling_on_sc=False),
        scratch_types=[
            pltpu.VMEM((NCH, ICH), jnp.int32),
            pltpu.VMEM((NCH, ICH), jnp.int32),
            pltpu.VMEM((BPW, D), jnp.float32),
            pltpu.VMEM((BPW, D), jnp.float32),
            pltpu.SemaphoreType.DMA,
        ],
    )
    def sc_gather(uidx_hbm, bidx_hbm, utab_hbm, btab_hbm, uout_hbm, bout_hbm,
                  uidx_v, bidx_v, urows_v, brows_v, sem):
        wid = lax.axis_index("s") * NC + lax.axis_index("c")
        base = wid * BPW
        pltpu.sync_copy(uidx_hbm.at[wid], uidx_v)
        pltpu.sync_copy(bidx_hbm.at[wid], bidx_v)
        copies = []
        for j in range(NCH):
            copies.append(pltpu.async_copy(
                utab_hbm.at[uidx_v.at[j]], urows_v.at[pl.ds(j * ICH, ICH)], sem))
            copies.append(pltpu.async_copy(
                btab_hbm.at[bidx_v.at[j]], brows_v.at[pl.ds(j * ICH, ICH)], sem))
        for c in copies:
            c.wait()
        pltpu.sync_copy(urows_v, uout_hbm.at[pl.ds(base, BPW)])
        pltpu.sync_copy(brows_v, bout_hbm.at[pl.ds(base, BPW)])

    return sc_gather


BLK = 2048


def _tc_body(tag_ref, u_ref, bk_ref, w_ref, b_ref, out_ref):
    proj = jnp.dot(tag_ref[...], w_ref[...],
                   preferred_element_type=jnp.float32) + b_ref[...]
    out_ref[...] = jnp.sum(u_ref[...] * (bk_ref[...] + proj), axis=1)


def _tc_combine(tag, u_emb, bk_emb, w_lin, b2d):
    return pl.pallas_call(
        _tc_body,
        grid=(B // BLK,),
        in_specs=[
            pl.BlockSpec((BLK, H), lambda i: (i, 0)),
            pl.BlockSpec((BLK, D), lambda i: (i, 0)),
            pl.BlockSpec((BLK, D), lambda i: (i, 0)),
            pl.BlockSpec((H, D), lambda i: (0, 0)),
            pl.BlockSpec((1, D), lambda i: (0, 0)),
        ],
        out_specs=pl.BlockSpec((BLK,), lambda i: (i,)),
        out_shape=jax.ShapeDtypeStruct((B,), jnp.float32),
    )(tag, u_emb, bk_emb, w_lin, b2d)


def kernel(user, book, tag_embedding, user_table, book_table, W_lin, b_lin):
    uidx = user.reshape(NW, NCH, ICH)
    bidx = book.reshape(NW, NCH, ICH)
    u_emb, bk_emb = _build_sc_gather()(uidx, bidx, user_table, book_table)
    return _tc_combine(tag_embedding, u_emb, bk_emb, W_lin,
                       b_lin.reshape(1, D))
